# baseline (speedup 1.0000x reference)
import jax
import jax.numpy as jnp
from jax import lax
from jax.experimental import pallas as pl
from jax.experimental.pallas import tpu as pltpu
from jax.experimental.pallas import tpu_sc as plsc

VOCAB = 1000000
EMBED_DIM = 32
BATCH = 16384
HIST = 50

NUM_WORKERS = 32
B_PER_W = BATCH // NUM_WORKERS
BB = 128
NQ = B_PER_W // BB
NSTEPS = HIST * NQ
NSLOT = 3
PACK = 128 // EMBED_DIM


def _glove_sc(idxT_hbm, packed_hbm, out_hbm, idx_v, off_v, g_v, tr_v,
              sem_g, sem_o):
    wid = lax.axis_index("s") * 2 + lax.axis_index("c")
    b0 = wid * B_PER_W

    pltpu.sync_copy(idxT_hbm.at[:, pl.ds(b0, B_PER_W)], idx_v)

    def split_body(i, _):
        t = i // (B_PER_W // 16)
        j = i % (B_PER_W // 16)
        v = idx_v[t, pl.ds(j * 16, 16)]
        off_v[t, pl.ds(j * 16, 16)] = (v & (PACK - 1)) * EMBED_DIM
        idx_v[t, pl.ds(j * 16, 16)] = v >> 2
        return ()

    lax.fori_loop(0, HIST * (B_PER_W // 16), split_body, (), unroll=False)

    lanes = lax.iota(jnp.int32, 16)

    def fire(k, slot):
        t = k // NQ
        q = k % NQ
        pltpu.async_copy(
            packed_hbm.at[idx_v.at[t, pl.ds(q * BB, BB)]],
            g_v.at[slot],
            sem_g.at[slot],
        )

    def gather_wait(slot):
        pltpu.make_async_copy(
            packed_hbm.at[pl.ds(0, BB)], g_v.at[slot], sem_g.at[slot]
        ).wait()

    def out_wait(slot):
        pltpu.make_async_copy(
            tr_v.at[slot, :, pl.ds(0, BB)],
            out_hbm.at[0, :, pl.ds(0, BB)],
            sem_o.at[slot],
        ).wait()

    def step(k, slot, first, fire_next):
        t = k // NQ
        q = k % NQ
        gather_wait(slot)

        @pl.when(jnp.logical_not(first))
        def _():
            out_wait(slot)

        def tr_body(j, _):
            offv = off_v[t, pl.ds(q * BB + 16 * j, 16)]
            for i in range(16):
                b = 16 * j + i
                offb = offv[i]
                bvec = jnp.zeros((16,), jnp.int32) + b
                for dblk in range(EMBED_DIM // 16):
                    val = g_v[slot, b, pl.ds(offb + 16 * dblk, 16)]
                    plsc.store_scatter(
                        tr_v.at[slot], [lanes + 16 * dblk, bvec], val
                    )
            return ()

        lax.fori_loop(0, BB // 16, tr_body, (), unroll=False)

        @pl.when(fire_next)
        def _():
            fire(k + NSLOT, slot)

        pltpu.async_copy(
            tr_v.at[slot, :, pl.ds(0, BB)],
            out_hbm.at[t, :, pl.ds(b0 + q * BB, BB)],
            sem_o.at[slot],
        )

    for slot in range(NSLOT):
        fire(slot, slot)

    def body(kk, _):
        for slot in range(NSLOT):
            k = NSLOT * kk + slot
            step(k, slot, kk == 0, k + NSLOT < NSTEPS)
        return ()

    lax.fori_loop(0, NSTEPS // NSLOT, body, (), unroll=False)

    for k in range(NSLOT * (NSTEPS // NSLOT), NSTEPS):
        step(jnp.int32(k), k % NSLOT, jnp.bool_(False), jnp.bool_(False))

    for slot in range(NSLOT):
        out_wait(slot)


@jax.jit
def kernel(idx, weights):
    mesh = plsc.VectorSubcoreMesh(core_axis_name="c", subcore_axis_name="s")
    packed = weights.reshape(VOCAB // PACK, 128)
    out_t = pl.kernel(
        _glove_sc,
        out_type=jax.ShapeDtypeStruct((HIST, EMBED_DIM, BATCH), jnp.float32),
        mesh=mesh,
        scratch_types=[
            pltpu.VMEM((HIST, B_PER_W), jnp.int32),
            pltpu.VMEM((HIST, B_PER_W), jnp.int32),
            pltpu.VMEM((NSLOT, BB, 128), jnp.float32),
            pltpu.VMEM((NSLOT, EMBED_DIM, BB + 8), jnp.float32),
            pltpu.SemaphoreType.DMA((NSLOT,)),
            pltpu.SemaphoreType.DMA((NSLOT,)),
        ],
        compiler_params=pltpu.CompilerParams(
            use_tc_tiling_on_sc=True, needs_layout_passes=False
        ),
    )(idx.T, packed)
    return jnp.transpose(out_t, (2, 0, 1))


# speedup vs baseline: 1.0098x; 1.0098x over previous
import jax
import jax.numpy as jnp
from jax import lax
from jax.experimental import pallas as pl
from jax.experimental.pallas import tpu as pltpu
from jax.experimental.pallas import tpu_sc as plsc

VOCAB = 1000000
EMBED_DIM = 32
BATCH = 16384
HIST = 50

NUM_WORKERS = 32
B_PER_W = BATCH // NUM_WORKERS
GB = 8
SLOTS = 4
NG = B_PER_W // GB
NOUT = NG // SLOTS


def _glove_sc(idx_hbm, table_hbm, out_hbm, idx_v, rows_v, sem_g, sem_o):
    wid = lax.axis_index("s") * 2 + lax.axis_index("c")
    b0 = wid * B_PER_W

    pltpu.sync_copy(idx_hbm.at[pl.ds(b0, B_PER_W)], idx_v)

    def fire(g, slot):
        for j in range(GB):
            pltpu.async_copy(
                table_hbm.at[idx_v.at[g * GB + j]],
                rows_v.at[slot, j],
                sem_g.at[slot],
            )

    def drain_gathers(slot):
        pltpu.make_async_copy(
            out_hbm.at[pl.ds(0, GB)], rows_v.at[slot], sem_g.at[slot]
        ).wait()

    def out_start(g, slot):
        pltpu.async_copy(
            rows_v.at[slot],
            out_hbm.at[pl.ds(b0 + g * GB, GB)],
            sem_o.at[slot],
        )

    def out_wait(slot):
        pltpu.make_async_copy(
            rows_v.at[slot], out_hbm.at[pl.ds(0, GB)], sem_o.at[slot]
        ).wait()

    for slot in range(SLOTS):
        fire(slot, slot)

    def body(i, _):
        gbase = i * SLOTS
        for slot in range(SLOTS):
            drain_gathers(slot)
            out_start(gbase + slot, slot)
        for slot in range(SLOTS):
            out_wait(slot)
            fire(gbase + slot + SLOTS, slot)
        return ()

    lax.fori_loop(0, NOUT - 1, body, (), unroll=False)

    gbase = (NOUT - 1) * SLOTS
    for slot in range(SLOTS):
        drain_gathers(slot)
        out_start(gbase + slot, slot)
    for slot in range(SLOTS):
        out_wait(slot)


@jax.jit
def kernel(idx, weights):
    mesh = plsc.VectorSubcoreMesh(core_axis_name="c", subcore_axis_name="s")
    return pl.kernel(
        _glove_sc,
        out_type=jax.ShapeDtypeStruct((BATCH, HIST, EMBED_DIM), jnp.float32),
        mesh=mesh,
        scratch_types=[
            pltpu.VMEM((B_PER_W, HIST), jnp.int32),
            pltpu.VMEM((SLOTS, GB, HIST, EMBED_DIM), jnp.float32),
            pltpu.SemaphoreType.DMA((SLOTS,)),
            pltpu.SemaphoreType.DMA((SLOTS,)),
        ],
        compiler_params=pltpu.CompilerParams(use_tc_tiling_on_sc=False),
    )(idx, weights)
